# baseline (speedup 1.0000x reference)
import functools

import jax
import jax.numpy as jnp
from jax import lax
from jax.experimental import pallas as pl
from jax.experimental.pallas import tpu as pltpu
from jax.experimental.pallas import tpu_sc as plsc

N_NODES = 10000
C = 128
K = 32
N_WORKERS = 32
NODES_PER_W = 320
N_PAD = N_WORKERS * NODES_PER_W
CHUNK_NODES = 2
CHUNKS = NODES_PER_W // CHUNK_NODES
NBUF = 4
LANES = 16
COLS = C // LANES

MM_BLOCK = 1280


def _mm_body(x_ref, w_ref, bias_ref, a_ref, b_ref):
    xb = x_ref[...]
    w1 = w_ref[:, :C]
    w2 = w_ref[:, C:]
    dn = (((1,), (1,)), ((), ()))
    a_ref[...] = lax.dot_general(xb, w1 - w2, dn,
                                 preferred_element_type=jnp.float32) + bias_ref[...]
    b_ref[...] = lax.dot_general(xb, w2, dn,
                                 preferred_element_type=jnp.float32)


@jax.jit
def _mm_call(x_pad, W, bias):
    grid = (N_PAD // MM_BLOCK,)
    return pl.pallas_call(
        _mm_body,
        grid=grid,
        in_specs=[
            pl.BlockSpec((MM_BLOCK, C), lambda i: (i, 0)),
            pl.BlockSpec((C, 2 * C), lambda i: (0, 0)),
            pl.BlockSpec((1, C), lambda i: (0, 0)),
        ],
        out_specs=[
            pl.BlockSpec((MM_BLOCK, C), lambda i: (i, 0)),
            pl.BlockSpec((MM_BLOCK, C), lambda i: (i, 0)),
        ],
        out_shape=[
            jax.ShapeDtypeStruct((N_PAD, C), jnp.float32),
            jax.ShapeDtypeStruct((N_PAD, C), jnp.float32),
        ],
    )(x_pad, W, bias)


def _sc_body(a_hbm, b_hbm, idx_hbm, out_hbm, idx_v, o_v, *rest):
    bufs = rest[:NBUF]
    sems = rest[NBUF:]
    wid = lax.axis_index("s") * 2 + lax.axis_index("c")
    base = wid * NODES_PER_W
    pltpu.sync_copy(idx_hbm.at[wid], idx_v)
    pltpu.sync_copy(a_hbm.at[pl.ds(base, NODES_PER_W)], o_v)

    def start(c, buf, sm):
        pltpu.make_async_copy(b_hbm.at[idx_v.at[c]], buf, sm).start()

    def wait(buf, sm):
        pltpu.make_async_copy(b_hbm.at[idx_v.at[0]], buf, sm).wait()

    def compute(c, buf):
        for n in range(CHUNK_NODES):
            row = c * CHUNK_NODES + n
            for col in range(COLS):
                sl = pl.ds(col * LANES, LANES)
                vals = [buf[n * K + k, sl] for k in range(K)]
                while len(vals) > 1:
                    vals = [jnp.maximum(vals[i], vals[i + 1])
                            for i in range(0, len(vals), 2)]
                z = o_v[row, sl] + vals[0]
                o_v[row, sl] = jnp.where(z > 0, z, jnp.exp(z) - 1.0)

    for c in range(NBUF - 1):
        start(c, bufs[c], sems[c])

    def group_body(i, carry):
        g = i * NBUF
        for par in range(NBUF):
            c = g + par
            pre = c + NBUF - 1
            pslot = (par + NBUF - 1) % NBUF
            @pl.when(pre < CHUNKS)
            def _():
                start(pre, bufs[pslot], sems[pslot])
            wait(bufs[par], sems[par])
            compute(c, bufs[par])
        return carry

    lax.fori_loop(0, CHUNKS // NBUF, group_body, 0)
    pltpu.sync_copy(o_v, out_hbm.at[pl.ds(base, NODES_PER_W)])


@jax.jit
def _sc_call(A, B, idx):
    mesh = plsc.VectorSubcoreMesh(core_axis_name="c", subcore_axis_name="s")
    f = functools.partial(
        pl.kernel,
        out_type=jax.ShapeDtypeStruct((N_PAD, C), jnp.float32),
        mesh=mesh,
        scratch_types=(
            [pltpu.VMEM((CHUNKS, CHUNK_NODES * K), jnp.int32),
             pltpu.VMEM((NODES_PER_W, C), jnp.float32)]
            + [pltpu.VMEM((CHUNK_NODES * K, C), jnp.float32)] * NBUF
            + [pltpu.SemaphoreType.DMA] * NBUF
        ),
    )(_sc_body)
    return f(A, B, idx)


def kernel(x, edge_index, W, b):
    x = x.astype(jnp.float32)
    ei = edge_index.astype(jnp.int32)
    x_pad = jnp.concatenate([x, jnp.zeros((N_PAD - N_NODES, C), jnp.float32)], axis=0)
    A, B = _mm_call(x_pad, W, b.reshape(1, C))
    ei_pad = jnp.concatenate(
        [ei, jnp.zeros((N_PAD - N_NODES, K), jnp.int32)], axis=0)
    idx = ei_pad.reshape(N_WORKERS, CHUNKS, CHUNK_NODES * K)
    out_pad = _sc_call(A, B, idx)
    return out_pad[:N_NODES]


# speedup vs baseline: 1.9938x; 1.9938x over previous
import functools

import jax
import jax.numpy as jnp
from jax import lax
from jax.experimental import pallas as pl
from jax.experimental.pallas import tpu as pltpu
from jax.experimental.pallas import tpu_sc as plsc

N_NODES = 10000
C = 128
K = 32
LANES = 16

N_SC = 2
N_TILES = 16
FPT = C // N_TILES
N_PAD = 10240
NODES_PER_SC = N_PAD // N_SC
CHUNK_N = 128
SC_CHUNKS = NODES_PER_SC // CHUNK_N
GROUPS = CHUNK_N // LANES

MM_BLOCK = 1280


def _mm_body(x_ref, w_ref, bias_ref, a_ref, bt_ref):
    xb = x_ref[...]
    w1 = w_ref[:, :C]
    w2 = w_ref[:, C:]
    dn = (((1,), (1,)), ((), ()))
    a_ref[...] = lax.dot_general(xb, w1 - w2, dn,
                                 preferred_element_type=jnp.float32) + bias_ref[...]
    bt_ref[...] = lax.dot_general(w2, xb, dn,
                                  preferred_element_type=jnp.float32)


@jax.jit
def _mm_call(x_pad, W, bias):
    grid = (N_PAD // MM_BLOCK,)
    return pl.pallas_call(
        _mm_body,
        grid=grid,
        in_specs=[
            pl.BlockSpec((MM_BLOCK, C), lambda i: (i, 0)),
            pl.BlockSpec((C, 2 * C), lambda i: (0, 0)),
            pl.BlockSpec((1, C), lambda i: (0, 0)),
        ],
        out_specs=[
            pl.BlockSpec((MM_BLOCK, C), lambda i: (i, 0)),
            pl.BlockSpec((C, MM_BLOCK), lambda i: (0, i)),
        ],
        out_shape=[
            jax.ShapeDtypeStruct((N_PAD, C), jnp.float32),
            jax.ShapeDtypeStruct((C, N_PAD), jnp.float32),
        ],
    )(x_pad, W, bias)


def _elu_body(a_ref, mt_ref, o_ref):
    z = a_ref[...] + mt_ref[...].T
    o_ref[...] = jnp.where(z > 0, z, jnp.exp(z) - 1.0)


@jax.jit
def _elu_call(A, Mt):
    grid = (N_PAD // MM_BLOCK,)
    return pl.pallas_call(
        _elu_body,
        grid=grid,
        in_specs=[
            pl.BlockSpec((MM_BLOCK, C), lambda i: (i, 0)),
            pl.BlockSpec((C, MM_BLOCK), lambda i: (0, i)),
        ],
        out_specs=pl.BlockSpec((MM_BLOCK, C), lambda i: (i, 0)),
        out_shape=jax.ShapeDtypeStruct((N_PAD, C), jnp.float32),
    )(A, Mt)


def _sc_body(bt_hbm, idx_hbm, mt_hbm, b_v,
             idx0, idx1, o0, o1, isem0, isem1, osem0, osem1):
    sc = lax.axis_index("c")
    tile = lax.axis_index("s")
    fbase = tile * FPT
    nbase = sc * NODES_PER_SC
    pltpu.sync_copy(bt_hbm.at[pl.ds(fbase, FPT)], b_v)

    idx_bufs = (idx0, idx1)
    idx_sems = (isem0, isem1)
    o_bufs = (o0, o1)
    o_sems = (osem0, osem1)

    def start_idx(ch, buf, sm):
        pltpu.make_async_copy(
            idx_hbm.at[pl.ds((nbase + ch * CHUNK_N) * K, CHUNK_N * K)],
            buf, sm).start()

    def wait_idx(buf, sm):
        pltpu.make_async_copy(
            idx_hbm.at[pl.ds(0, CHUNK_N * K)], buf, sm).wait()

    def wait_store(buf, sm):
        pltpu.make_async_copy(
            buf, mt_hbm.at[pl.ds(0, FPT), pl.ds(0, CHUNK_N)], sm).wait()

    iota = lax.iota(jnp.int32, LANES)

    def compute_chunk(idx_v, o_v):
        def group_body(g, carry):
            nkbase = (iota + g * LANES) * K
            accs = [None] * FPT
            for k in range(K):
                nid = plsc.load_gather(idx_v, [nkbase + k])
                for f in range(FPT):
                    v = plsc.load_gather(
                        b_v, [jnp.full((LANES,), f, jnp.int32), nid])
                    accs[f] = v if k == 0 else jnp.maximum(accs[f], v)
            for f in range(FPT):
                o_v[f, pl.ds(g * LANES, LANES)] = accs[f]
            return carry
        lax.fori_loop(0, GROUPS, group_body, 0)

    start_idx(0, idx_bufs[0], idx_sems[0])

    def pair_body(i, carry):
        base = i * 2
        for par in range(2):
            ch = base + par
            @pl.when(ch + 1 < SC_CHUNKS)
            def _():
                start_idx(ch + 1, idx_bufs[1 - par], idx_sems[1 - par])
            wait_idx(idx_bufs[par], idx_sems[par])
            @pl.when(ch >= 2)
            def _():
                wait_store(o_bufs[par], o_sems[par])
            compute_chunk(idx_bufs[par], o_bufs[par])
            pltpu.make_async_copy(
                o_bufs[par],
                mt_hbm.at[pl.ds(fbase, FPT),
                          pl.ds(nbase + ch * CHUNK_N, CHUNK_N)],
                o_sems[par]).start()
        return carry

    lax.fori_loop(0, SC_CHUNKS // 2, pair_body, 0)
    wait_store(o_bufs[0], o_sems[0])
    wait_store(o_bufs[1], o_sems[1])


@jax.jit
def _sc_call(Bt, idx):
    mesh = plsc.VectorSubcoreMesh(core_axis_name="c", subcore_axis_name="s")
    f = functools.partial(
        pl.kernel,
        out_type=jax.ShapeDtypeStruct((C, N_PAD), jnp.float32),
        mesh=mesh,
        compiler_params=pltpu.CompilerParams(needs_layout_passes=False),
        scratch_types=(
            [pltpu.VMEM((FPT, N_PAD), jnp.float32)]
            + [pltpu.VMEM((CHUNK_N * K,), jnp.int32)] * 2
            + [pltpu.VMEM((FPT, CHUNK_N), jnp.float32)] * 2
            + [pltpu.SemaphoreType.DMA] * 4
        ),
    )(_sc_body)
    return f(Bt, idx)


def kernel(x, edge_index, W, b):
    x = x.astype(jnp.float32)
    ei = edge_index.astype(jnp.int32)
    x_pad = jnp.concatenate([x, jnp.zeros((N_PAD - N_NODES, C), jnp.float32)], axis=0)
    A, Bt = _mm_call(x_pad, W, b.reshape(1, C))
    ei_pad = jnp.concatenate(
        [ei, jnp.zeros((N_PAD - N_NODES, K), jnp.int32)], axis=0)
    Mt = _sc_call(Bt, ei_pad.reshape(-1))
    out_pad = _elu_call(A, Mt)
    return out_pad[:N_NODES]


# speedup vs baseline: 3.0190x; 1.5142x over previous
import functools

import jax
import jax.numpy as jnp
from jax import lax
from jax.experimental import pallas as pl
from jax.experimental.pallas import tpu as pltpu
from jax.experimental.pallas import tpu_sc as plsc

N_NODES = 10000
C = 128
K = 32
LANES = 16

N_SC = 2
N_TILES = 16
FPT = C // N_TILES
N_PAD = 10240
NODES_PER_SC = N_PAD // N_SC
CHUNK_N = 128
SC_CHUNKS = NODES_PER_SC // CHUNK_N
GROUPS = CHUNK_N // LANES

MM_BLOCK = 1280


def _mm_body(x_ref, w_ref, bias_ref, a_ref, bt_ref):
    xb = x_ref[...]
    w1 = w_ref[:, :C]
    w2 = w_ref[:, C:]
    dn = (((1,), (1,)), ((), ()))
    a_ref[...] = lax.dot_general(xb, w1 - w2, dn,
                                 preferred_element_type=jnp.float32) + bias_ref[...]
    bt_ref[...] = lax.dot_general(w2, xb, dn,
                                  preferred_element_type=jnp.float32)


@jax.jit
def _mm_call(x_pad, W, bias):
    grid = (N_PAD // MM_BLOCK,)
    return pl.pallas_call(
        _mm_body,
        grid=grid,
        in_specs=[
            pl.BlockSpec((MM_BLOCK, C), lambda i: (i, 0)),
            pl.BlockSpec((C, 2 * C), lambda i: (0, 0)),
            pl.BlockSpec((1, C), lambda i: (0, 0)),
        ],
        out_specs=[
            pl.BlockSpec((MM_BLOCK, C), lambda i: (i, 0)),
            pl.BlockSpec((C, MM_BLOCK), lambda i: (0, i)),
        ],
        out_shape=[
            jax.ShapeDtypeStruct((N_PAD, C), jnp.float32),
            jax.ShapeDtypeStruct((C, N_PAD), jnp.float32),
        ],
    )(x_pad, W, bias)


def _elu_body(a_ref, mt_ref, o_ref):
    z = a_ref[...] + mt_ref[...].T
    o_ref[...] = jnp.where(z > 0, z, jnp.exp(z) - 1.0)


@jax.jit
def _elu_call(A, Mt):
    grid = (N_PAD // MM_BLOCK,)
    return pl.pallas_call(
        _elu_body,
        grid=grid,
        in_specs=[
            pl.BlockSpec((MM_BLOCK, C), lambda i: (i, 0)),
            pl.BlockSpec((C, MM_BLOCK), lambda i: (0, i)),
        ],
        out_specs=pl.BlockSpec((MM_BLOCK, C), lambda i: (i, 0)),
        out_shape=jax.ShapeDtypeStruct((N_PAD, C), jnp.float32),
    )(A, Mt)


def _sc_body(bt_hbm, idx_hbm, mt_hbm, b_v,
             idx0, idx1, o0, o1, isem0, isem1, osem0, osem1):
    sc = lax.axis_index("c")
    tile = lax.axis_index("s")
    fbase = tile * FPT
    nbase = sc * NODES_PER_SC
    pltpu.sync_copy(bt_hbm.at[pl.ds(fbase, FPT)], b_v)

    idx_bufs = (idx0, idx1)
    idx_sems = (isem0, isem1)
    o_bufs = (o0, o1)
    o_sems = (osem0, osem1)

    def start_idx(ch, buf, sm):
        pltpu.make_async_copy(
            idx_hbm.at[pl.ds((nbase + ch * CHUNK_N) * K, CHUNK_N * K)],
            buf, sm).start()

    def wait_idx(buf, sm):
        pltpu.make_async_copy(
            idx_hbm.at[pl.ds(0, CHUNK_N * K)], buf, sm).wait()

    def wait_store(buf, sm):
        pltpu.make_async_copy(
            buf, mt_hbm.at[pl.ds(0, FPT), pl.ds(0, CHUNK_N)], sm).wait()

    iota = lax.iota(jnp.int32, LANES)

    def compute_chunk(idx_v, o_v):
        def group_body(g, carry):
            gbase = g * (K * LANES)
            accs = [None] * FPT
            for k in range(K):
                nid = idx_v[pl.ds(gbase + k * LANES, LANES)]
                for f in range(FPT):
                    v = plsc.load_gather(
                        b_v, [jnp.full((LANES,), f, jnp.int32), nid])
                    accs[f] = v if k == 0 else jnp.maximum(accs[f], v)
            for f in range(FPT):
                o_v[f, pl.ds(g * LANES, LANES)] = accs[f]
            return carry
        lax.fori_loop(0, GROUPS, group_body, 0)

    start_idx(0, idx_bufs[0], idx_sems[0])

    def pair_body(i, carry):
        base = i * 2
        for par in range(2):
            ch = base + par
            @pl.when(ch + 1 < SC_CHUNKS)
            def _():
                start_idx(ch + 1, idx_bufs[1 - par], idx_sems[1 - par])
            wait_idx(idx_bufs[par], idx_sems[par])
            @pl.when(ch >= 2)
            def _():
                wait_store(o_bufs[par], o_sems[par])
            compute_chunk(idx_bufs[par], o_bufs[par])
            pltpu.make_async_copy(
                o_bufs[par],
                mt_hbm.at[pl.ds(fbase, FPT),
                          pl.ds(nbase + ch * CHUNK_N, CHUNK_N)],
                o_sems[par]).start()
        return carry

    lax.fori_loop(0, SC_CHUNKS // 2, pair_body, 0)
    wait_store(o_bufs[0], o_sems[0])
    wait_store(o_bufs[1], o_sems[1])


@jax.jit
def _sc_call(Bt, idx):
    mesh = plsc.VectorSubcoreMesh(core_axis_name="c", subcore_axis_name="s")
    f = functools.partial(
        pl.kernel,
        out_type=jax.ShapeDtypeStruct((C, N_PAD), jnp.float32),
        mesh=mesh,
        compiler_params=pltpu.CompilerParams(needs_layout_passes=False),
        scratch_types=(
            [pltpu.VMEM((FPT, N_PAD), jnp.float32)]
            + [pltpu.VMEM((CHUNK_N * K,), jnp.int32)] * 2
            + [pltpu.VMEM((FPT, CHUNK_N), jnp.float32)] * 2
            + [pltpu.SemaphoreType.DMA] * 4
        ),
    )(_sc_body)
    return f(Bt, idx)


def kernel(x, edge_index, W, b):
    x = x.astype(jnp.float32)
    ei = edge_index.astype(jnp.int32)
    x_pad = jnp.concatenate([x, jnp.zeros((N_PAD - N_NODES, C), jnp.float32)], axis=0)
    A, Bt = _mm_call(x_pad, W, b.reshape(1, C))
    ei_pad = jnp.concatenate(
        [ei, jnp.zeros((N_PAD - N_NODES, K), jnp.int32)], axis=0)
    ei_t = ei_pad.reshape(N_PAD // LANES, LANES, K).transpose(0, 2, 1)
    Mt = _sc_call(Bt, ei_t.reshape(-1))
    out_pad = _elu_call(A, Mt)
    return out_pad[:N_NODES]


# speedup vs baseline: 4.8460x; 1.6052x over previous
import functools

import jax
import jax.numpy as jnp
from jax import lax
from jax.experimental import pallas as pl
from jax.experimental.pallas import tpu as pltpu
from jax.experimental.pallas import tpu_sc as plsc

N_NODES = 10000
C = 128
K = 32
LANES = 16

N_SC = 2
N_TILES = 16
FPT = C // N_TILES
N_PAD = 10240
NODES_PER_SC = N_PAD // N_SC
CHUNK_N = 128
SC_CHUNKS = NODES_PER_SC // CHUNK_N
GROUPS = CHUNK_N // LANES

MM_BLOCK = 1280


def _mm_body(x_ref, w_ref, bias_ref, a_ref, bp_ref):
    xb = x_ref[...]
    w1 = w_ref[:, :C]
    w2 = w_ref[:, C:]
    dn = (((1,), (1,)), ((), ()))
    a_ref[...] = lax.dot_general(xb, w1 - w2, dn,
                                 preferred_element_type=jnp.float32) + bias_ref[...]
    btf = lax.dot_general(w2, xb, dn, preferred_element_type=jnp.float32)
    bb = btf.astype(jnp.bfloat16)
    lo = lax.bitcast_convert_type(bb[:C // 2, :], jnp.uint16).astype(jnp.uint32)
    hi = lax.bitcast_convert_type(bb[C // 2:, :], jnp.uint16).astype(jnp.uint32)
    bp_ref[...] = lax.bitcast_convert_type(lo | (hi << 16), jnp.int32)


@jax.jit
def _mm_call(x_pad, W, bias):
    grid = (N_PAD // MM_BLOCK,)
    return pl.pallas_call(
        _mm_body,
        grid=grid,
        in_specs=[
            pl.BlockSpec((MM_BLOCK, C), lambda i: (i, 0)),
            pl.BlockSpec((C, 2 * C), lambda i: (0, 0)),
            pl.BlockSpec((1, C), lambda i: (0, 0)),
        ],
        out_specs=[
            pl.BlockSpec((MM_BLOCK, C), lambda i: (i, 0)),
            pl.BlockSpec((C // 2, MM_BLOCK), lambda i: (0, i)),
        ],
        out_shape=[
            jax.ShapeDtypeStruct((N_PAD, C), jnp.float32),
            jax.ShapeDtypeStruct((C // 2, N_PAD), jnp.int32),
        ],
    )(x_pad, W, bias)


def _elu_body(a_ref, mt_ref, o_ref):
    z = a_ref[...] + mt_ref[...].T
    o_ref[...] = jnp.where(z > 0, z, jnp.exp(z) - 1.0)


@jax.jit
def _elu_call(A, Mt):
    grid = (N_PAD // MM_BLOCK,)
    return pl.pallas_call(
        _elu_body,
        grid=grid,
        in_specs=[
            pl.BlockSpec((MM_BLOCK, C), lambda i: (i, 0)),
            pl.BlockSpec((C, MM_BLOCK), lambda i: (0, i)),
        ],
        out_specs=pl.BlockSpec((MM_BLOCK, C), lambda i: (i, 0)),
        out_shape=jax.ShapeDtypeStruct((N_PAD, C), jnp.float32),
    )(A, Mt)


def _sc_body(bt_hbm, idx_hbm, mt_hbm, b_v,
             idx0, idx1, o0, o1, isem0, isem1, osem0, osem1):
    sc = lax.axis_index("c")
    tile = lax.axis_index("s")
    nbase = sc * NODES_PER_SC
    pltpu.sync_copy(bt_hbm.at[pl.ds(tile * (FPT // 2), FPT // 2)], b_v)

    idx_bufs = (idx0, idx1)
    idx_sems = (isem0, isem1)
    o_bufs = (o0, o1)
    o_sems = (osem0, osem1)

    def start_idx(ch, buf, sm):
        pltpu.make_async_copy(
            idx_hbm.at[pl.ds((nbase + ch * CHUNK_N) * K, CHUNK_N * K)],
            buf, sm).start()

    def wait_idx(buf, sm):
        pltpu.make_async_copy(
            idx_hbm.at[pl.ds(0, CHUNK_N * K)], buf, sm).wait()

    def wait_store(buf, sm):
        pltpu.make_async_copy(
            buf, mt_hbm.at[pl.ds(0, FPT), pl.ds(0, CHUNK_N)], sm).wait()

    iota = lax.iota(jnp.int32, LANES)

    def compute_chunk(idx_v, o_v):
        def group_body(g, carry):
            gbase = g * (K * LANES)
            accs = [None] * (FPT // 2)
            for k in range(K):
                nid = idx_v[pl.ds(gbase + k * LANES, LANES)]
                for fp in range(FPT // 2):
                    w = plsc.load_gather(
                        b_v, [jnp.full((LANES,), fp, jnp.int32), nid])
                    v = plsc.bitcast(w, jnp.bfloat16)
                    accs[fp] = v if k == 0 else jnp.maximum(accs[fp], v)
            for fp in range(FPT // 2):
                lo_f, hi_f = plsc.unpack(
                    accs[fp], format=plsc.PackFormat.INTERLEAVED)
                o_v[fp, pl.ds(g * LANES, LANES)] = lo_f
                o_v[FPT // 2 + fp, pl.ds(g * LANES, LANES)] = hi_f
            return carry
        lax.fori_loop(0, GROUPS, group_body, 0)

    start_idx(0, idx_bufs[0], idx_sems[0])

    def pair_body(i, carry):
        base = i * 2
        for par in range(2):
            ch = base + par
            @pl.when(ch + 1 < SC_CHUNKS)
            def _():
                start_idx(ch + 1, idx_bufs[1 - par], idx_sems[1 - par])
            wait_idx(idx_bufs[par], idx_sems[par])
            @pl.when(ch >= 2)
            def _():
                wait_store(o_bufs[par], o_sems[par])
            compute_chunk(idx_bufs[par], o_bufs[par])
            nsl = pl.ds(nbase + ch * CHUNK_N, CHUNK_N)
            pltpu.make_async_copy(
                o_bufs[par].at[pl.ds(0, FPT // 2)],
                mt_hbm.at[pl.ds(tile * (FPT // 2), FPT // 2), nsl],
                o_sems[par]).start()
            pltpu.make_async_copy(
                o_bufs[par].at[pl.ds(FPT // 2, FPT // 2)],
                mt_hbm.at[pl.ds(C // 2 + tile * (FPT // 2), FPT // 2), nsl],
                o_sems[par]).start()
        return carry

    lax.fori_loop(0, SC_CHUNKS // 2, pair_body, 0)
    wait_store(o_bufs[0], o_sems[0])
    wait_store(o_bufs[1], o_sems[1])


@jax.jit
def _sc_call(Bt, idx):
    mesh = plsc.VectorSubcoreMesh(core_axis_name="c", subcore_axis_name="s")
    f = functools.partial(
        pl.kernel,
        out_type=jax.ShapeDtypeStruct((C, N_PAD), jnp.float32),
        mesh=mesh,
        compiler_params=pltpu.CompilerParams(needs_layout_passes=False),
        scratch_types=(
            [pltpu.VMEM((FPT // 2, N_PAD), jnp.int32)]
            + [pltpu.VMEM((CHUNK_N * K,), jnp.int32)] * 2
            + [pltpu.VMEM((FPT, CHUNK_N), jnp.float32)] * 2
            + [pltpu.SemaphoreType.DMA] * 4
        ),
    )(_sc_body)
    return f(Bt, idx)


def kernel(x, edge_index, W, b):
    x = x.astype(jnp.float32)
    ei = edge_index.astype(jnp.int32)
    x_pad = jnp.concatenate([x, jnp.zeros((N_PAD - N_NODES, C), jnp.float32)], axis=0)
    A, Bt = _mm_call(x_pad, W, b.reshape(1, C))
    ei_pad = jnp.concatenate(
        [ei, jnp.zeros((N_PAD - N_NODES, K), jnp.int32)], axis=0)
    ei_t = ei_pad.reshape(N_PAD // LANES, LANES, K).transpose(0, 2, 1)
    Mt = _sc_call(Bt, ei_t.reshape(-1))
    out_pad = _elu_call(A, Mt)
    return out_pad[:N_NODES]


# speedup vs baseline: 5.2084x; 1.0748x over previous
import functools

import jax
import jax.numpy as jnp
from jax import lax
from jax.experimental import pallas as pl
from jax.experimental.pallas import tpu as pltpu
from jax.experimental.pallas import tpu_sc as plsc

N_NODES = 10000
C = 128
K = 32
LANES = 16

N_SC = 2
N_TILES = 16
FPT = C // N_TILES
N_PAD = 10240
NODES_PER_SC = N_PAD // N_SC
CHUNK_N = 128
SC_CHUNKS = NODES_PER_SC // CHUNK_N
GROUPS = CHUNK_N // LANES

MM_BLOCK = 1280


def _mm_body(x_ref, w_ref, bias_ref, a_ref, bp_ref):
    xb = x_ref[...]
    w1 = w_ref[:, :C]
    w2 = w_ref[:, C:]
    dn = (((1,), (1,)), ((), ()))
    a_ref[...] = lax.dot_general(xb, w1 - w2, dn,
                                 preferred_element_type=jnp.float32) + bias_ref[...]
    btf = lax.dot_general(w2, xb, dn, preferred_element_type=jnp.float32)
    bb = btf.astype(jnp.bfloat16)
    lo = lax.bitcast_convert_type(bb[:C // 2, :], jnp.uint16).astype(jnp.uint32)
    hi = lax.bitcast_convert_type(bb[C // 2:, :], jnp.uint16).astype(jnp.uint32)
    bp_ref[...] = lax.bitcast_convert_type(lo | (hi << 16), jnp.int32)


@jax.jit
def _mm_call(x, W, bias):
    grid = (N_PAD // MM_BLOCK,)
    return pl.pallas_call(
        _mm_body,
        grid=grid,
        in_specs=[
            pl.BlockSpec((MM_BLOCK, C), lambda i: (i, 0)),
            pl.BlockSpec((C, 2 * C), lambda i: (0, 0)),
            pl.BlockSpec((1, C), lambda i: (0, 0)),
        ],
        out_specs=[
            pl.BlockSpec((MM_BLOCK, C), lambda i: (i, 0)),
            pl.BlockSpec((C // 2, MM_BLOCK), lambda i: (0, i)),
        ],
        out_shape=[
            jax.ShapeDtypeStruct((N_PAD, C), jnp.float32),
            jax.ShapeDtypeStruct((C // 2, N_PAD), jnp.int32),
        ],
    )(x, W, bias)


def _elu_body(a_ref, mt_ref, o_ref):
    z = a_ref[...] + mt_ref[...].T
    o_ref[...] = jnp.where(z > 0, z, jnp.exp(z) - 1.0)


@jax.jit
def _elu_call(A, Mt):
    grid = (N_PAD // MM_BLOCK,)
    return pl.pallas_call(
        _elu_body,
        grid=grid,
        in_specs=[
            pl.BlockSpec((MM_BLOCK, C), lambda i: (i, 0)),
            pl.BlockSpec((C, MM_BLOCK), lambda i: (0, i)),
        ],
        out_specs=pl.BlockSpec((MM_BLOCK, C), lambda i: (i, 0)),
        out_shape=jax.ShapeDtypeStruct((N_NODES, C), jnp.float32),
    )(A, Mt)


def _sc_body(bt_hbm, idx_hbm, mt_hbm, b_v,
             idx0, idx1, o0, o1, isem0, isem1, osem0, osem1):
    sc = lax.axis_index("c")
    tile = lax.axis_index("s")
    nbase = sc * NODES_PER_SC
    pltpu.sync_copy(bt_hbm.at[pl.ds(tile * (FPT // 2), FPT // 2)], b_v)

    idx_bufs = (idx0, idx1)
    idx_sems = (isem0, isem1)
    o_bufs = (o0, o1)
    o_sems = (osem0, osem1)

    def start_idx(ch, buf, sm):
        pltpu.make_async_copy(
            idx_hbm.at[pl.ds((nbase + ch * CHUNK_N) * K, CHUNK_N * K)],
            buf, sm).start()

    def wait_idx(buf, sm):
        pltpu.make_async_copy(
            idx_hbm.at[pl.ds(0, CHUNK_N * K)], buf, sm).wait()

    def wait_store(buf, sm):
        pltpu.make_async_copy(
            buf, mt_hbm.at[pl.ds(0, FPT), pl.ds(0, CHUNK_N)], sm).wait()

    iota = lax.iota(jnp.int32, LANES)

    def compute_chunk(idx_v, o_v):
        def group_body(g, carry):
            gbase = g * (K * LANES)
            accs = [None] * (FPT // 2)
            for k in range(K):
                nid = idx_v[pl.ds(gbase + k * LANES, LANES)]
                for fp in range(FPT // 2):
                    w = plsc.load_gather(
                        b_v, [jnp.full((LANES,), fp, jnp.int32), nid])
                    v = plsc.bitcast(w, jnp.bfloat16)
                    accs[fp] = v if k == 0 else jnp.maximum(accs[fp], v)
            for fp in range(FPT // 2):
                lo_f, hi_f = plsc.unpack(
                    accs[fp], format=plsc.PackFormat.INTERLEAVED)
                o_v[fp, pl.ds(g * LANES, LANES)] = lo_f
                o_v[FPT // 2 + fp, pl.ds(g * LANES, LANES)] = hi_f
            return carry
        lax.fori_loop(0, GROUPS, group_body, 0)

    start_idx(0, idx_bufs[0], idx_sems[0])

    def pair_body(i, carry):
        base = i * 2
        for par in range(2):
            ch = base + par
            @pl.when(ch + 1 < SC_CHUNKS)
            def _():
                start_idx(ch + 1, idx_bufs[1 - par], idx_sems[1 - par])
            wait_idx(idx_bufs[par], idx_sems[par])
            @pl.when(ch >= 2)
            def _():
                wait_store(o_bufs[par], o_sems[par])
            compute_chunk(idx_bufs[par], o_bufs[par])
            nsl = pl.ds(nbase + ch * CHUNK_N, CHUNK_N)
            pltpu.make_async_copy(
                o_bufs[par].at[pl.ds(0, FPT // 2)],
                mt_hbm.at[pl.ds(tile * (FPT // 2), FPT // 2), nsl],
                o_sems[par]).start()
            pltpu.make_async_copy(
                o_bufs[par].at[pl.ds(FPT // 2, FPT // 2)],
                mt_hbm.at[pl.ds(C // 2 + tile * (FPT // 2), FPT // 2), nsl],
                o_sems[par]).start()
        return carry

    lax.fori_loop(0, SC_CHUNKS // 2, pair_body, 0)
    wait_store(o_bufs[0], o_sems[0])
    wait_store(o_bufs[1], o_sems[1])


@jax.jit
def _sc_call(Bt, idx):
    mesh = plsc.VectorSubcoreMesh(core_axis_name="c", subcore_axis_name="s")
    f = functools.partial(
        pl.kernel,
        out_type=jax.ShapeDtypeStruct((C, N_PAD), jnp.float32),
        mesh=mesh,
        compiler_params=pltpu.CompilerParams(needs_layout_passes=False),
        scratch_types=(
            [pltpu.VMEM((FPT // 2, N_PAD), jnp.int32)]
            + [pltpu.VMEM((CHUNK_N * K,), jnp.int32)] * 2
            + [pltpu.VMEM((FPT, CHUNK_N), jnp.float32)] * 2
            + [pltpu.SemaphoreType.DMA] * 4
        ),
    )(_sc_body)
    return f(Bt, idx)


def kernel(x, edge_index, W, b):
    x = x.astype(jnp.float32)
    ei = edge_index.astype(jnp.int32)
    A, Bp = _mm_call(x, W, b.reshape(1, C))
    ei_pad = jnp.concatenate(
        [ei, jnp.zeros((N_PAD - N_NODES, K), jnp.int32)], axis=0)
    ei_t = ei_pad.reshape(N_PAD // LANES, LANES, K).transpose(0, 2, 1)
    Mt = _sc_call(Bp, ei_t.reshape(-1))
    return _elu_call(A, Mt)


# speedup vs baseline: 5.2587x; 1.0097x over previous
import functools

import jax
import jax.numpy as jnp
from jax import lax
from jax.experimental import pallas as pl
from jax.experimental.pallas import tpu as pltpu
from jax.experimental.pallas import tpu_sc as plsc

N_NODES = 10000
C = 128
K = 32
LANES = 16

N_SC = 2
N_TILES = 16
FPT = C // N_TILES
N_PAD = 10240
NODES_PER_SC = N_PAD // N_SC
CHUNK_N = 128
SC_CHUNKS = NODES_PER_SC // CHUNK_N
GROUPS = CHUNK_N // LANES

MM_BLOCK = 1280


def _mm_body(x_ref, w_ref, bias_ref, a_ref, bp_ref):
    xb = x_ref[...]
    w1 = w_ref[:, :C]
    w2 = w_ref[:, C:]
    dn = (((1,), (1,)), ((), ()))
    a_ref[...] = (lax.dot_general(xb, w1 - w2, dn,
                                  preferred_element_type=jnp.float32)
                  + bias_ref[...]).astype(jnp.bfloat16)
    btf = lax.dot_general(w2, xb, dn, preferred_element_type=jnp.float32)
    bb = btf.astype(jnp.bfloat16)
    lo = lax.bitcast_convert_type(bb[:C // 2, :], jnp.uint16).astype(jnp.uint32)
    hi = lax.bitcast_convert_type(bb[C // 2:, :], jnp.uint16).astype(jnp.uint32)
    bp_ref[...] = lax.bitcast_convert_type(lo | (hi << 16), jnp.int32)


@jax.jit
def _mm_call(x, W, bias):
    grid = (N_PAD // MM_BLOCK,)
    return pl.pallas_call(
        _mm_body,
        grid=grid,
        in_specs=[
            pl.BlockSpec((MM_BLOCK, C), lambda i: (i, 0)),
            pl.BlockSpec((C, 2 * C), lambda i: (0, 0)),
            pl.BlockSpec((1, C), lambda i: (0, 0)),
        ],
        out_specs=[
            pl.BlockSpec((MM_BLOCK, C), lambda i: (i, 0)),
            pl.BlockSpec((C // 2, MM_BLOCK), lambda i: (0, i)),
        ],
        out_shape=[
            jax.ShapeDtypeStruct((N_PAD, C), jnp.bfloat16),
            jax.ShapeDtypeStruct((C // 2, N_PAD), jnp.int32),
        ],
    )(x, W, bias)


def _elu_body(a_ref, mt_ref, o_ref):
    w = lax.bitcast_convert_type(mt_ref[...], jnp.uint32)
    lo = lax.bitcast_convert_type(
        (w & 0xFFFF).astype(jnp.uint16), jnp.bfloat16).astype(jnp.float32)
    hi = lax.bitcast_convert_type(
        (w >> 16).astype(jnp.uint16), jnp.bfloat16).astype(jnp.float32)
    m = jnp.concatenate([lo, hi], axis=0)
    z = a_ref[...].astype(jnp.float32) + m.T
    o_ref[...] = jnp.where(z > 0, z, jnp.exp(z) - 1.0)


@jax.jit
def _elu_call(A, Mt):
    grid = (N_PAD // MM_BLOCK,)
    return pl.pallas_call(
        _elu_body,
        grid=grid,
        in_specs=[
            pl.BlockSpec((MM_BLOCK, C), lambda i: (i, 0)),
            pl.BlockSpec((C // 2, MM_BLOCK), lambda i: (0, i)),
        ],
        out_specs=pl.BlockSpec((MM_BLOCK, C), lambda i: (i, 0)),
        out_shape=jax.ShapeDtypeStruct((N_NODES, C), jnp.float32),
    )(A, Mt)


def _sc_body(bt_hbm, idx_hbm, mt_hbm, b_v,
             idx0, idx1, o0, o1, isem0, isem1, osem0, osem1):
    sc = lax.axis_index("c")
    tile = lax.axis_index("s")
    nbase = sc * NODES_PER_SC
    pltpu.sync_copy(bt_hbm.at[pl.ds(tile * (FPT // 2), FPT // 2)], b_v)

    idx_bufs = (idx0, idx1)
    idx_sems = (isem0, isem1)
    o_bufs = (o0, o1)
    o_sems = (osem0, osem1)

    def start_idx(ch, buf, sm):
        pltpu.make_async_copy(
            idx_hbm.at[pl.ds((nbase + ch * CHUNK_N) * K, CHUNK_N * K)],
            buf, sm).start()

    def wait_idx(buf, sm):
        pltpu.make_async_copy(
            idx_hbm.at[pl.ds(0, CHUNK_N * K)], buf, sm).wait()

    def wait_store(buf, sm):
        pltpu.make_async_copy(
            buf, mt_hbm.at[pl.ds(0, FPT // 2), pl.ds(0, CHUNK_N)], sm).wait()

    iota = lax.iota(jnp.int32, LANES)

    def compute_chunk(idx_v, o_v):
        def group_body(g, carry):
            gbase = g * (K * LANES)
            accs = [None] * (FPT // 2)
            for k in range(K):
                nid = idx_v[pl.ds(gbase + k * LANES, LANES)]
                for fp in range(FPT // 2):
                    w = plsc.load_gather(
                        b_v, [jnp.full((LANES,), fp, jnp.int32), nid])
                    v = plsc.bitcast(w, jnp.bfloat16)
                    accs[fp] = v if k == 0 else jnp.maximum(accs[fp], v)
            for fp in range(FPT // 2):
                o_v[fp, pl.ds(g * LANES, LANES)] = plsc.bitcast(
                    accs[fp], jnp.int32)
            return carry
        lax.fori_loop(0, GROUPS, group_body, 0)

    start_idx(0, idx_bufs[0], idx_sems[0])

    def pair_body(i, carry):
        base = i * 2
        for par in range(2):
            ch = base + par
            @pl.when(ch + 1 < SC_CHUNKS)
            def _():
                start_idx(ch + 1, idx_bufs[1 - par], idx_sems[1 - par])
            wait_idx(idx_bufs[par], idx_sems[par])
            @pl.when(ch >= 2)
            def _():
                wait_store(o_bufs[par], o_sems[par])
            compute_chunk(idx_bufs[par], o_bufs[par])
            pltpu.make_async_copy(
                o_bufs[par],
                mt_hbm.at[pl.ds(tile * (FPT // 2), FPT // 2),
                          pl.ds(nbase + ch * CHUNK_N, CHUNK_N)],
                o_sems[par]).start()
        return carry

    lax.fori_loop(0, SC_CHUNKS // 2, pair_body, 0)
    wait_store(o_bufs[0], o_sems[0])
    wait_store(o_bufs[1], o_sems[1])


@jax.jit
def _sc_call(Bt, idx):
    mesh = plsc.VectorSubcoreMesh(core_axis_name="c", subcore_axis_name="s")
    f = functools.partial(
        pl.kernel,
        out_type=jax.ShapeDtypeStruct((C // 2, N_PAD), jnp.int32),
        mesh=mesh,
        compiler_params=pltpu.CompilerParams(needs_layout_passes=False),
        scratch_types=(
            [pltpu.VMEM((FPT // 2, N_PAD), jnp.int32)]
            + [pltpu.VMEM((CHUNK_N * K,), jnp.int32)] * 2
            + [pltpu.VMEM((FPT // 2, CHUNK_N), jnp.int32)] * 2
            + [pltpu.SemaphoreType.DMA] * 4
        ),
    )(_sc_body)
    return f(Bt, idx)


def kernel(x, edge_index, W, b):
    x = x.astype(jnp.float32)
    ei = edge_index.astype(jnp.int32)
    A, Bp = _mm_call(x, W, b.reshape(1, C))
    ei_pad = jnp.concatenate(
        [ei, jnp.zeros((N_PAD - N_NODES, K), jnp.int32)], axis=0)
    ei_t = ei_pad.reshape(N_PAD // LANES, LANES, K).transpose(0, 2, 1)
    Mt = _sc_call(Bp, ei_t.reshape(-1))
    return _elu_call(A, Mt)


# speedup vs baseline: 6.2973x; 1.1975x over previous
import functools

import jax
import jax.numpy as jnp
from jax import lax
from jax.experimental import pallas as pl
from jax.experimental.pallas import tpu as pltpu
from jax.experimental.pallas import tpu_sc as plsc

N_NODES = 10000
C = 128
K = 32
LANES = 16

N_SC = 2
N_TILES = 16
FPT = C // N_TILES
N_PAD = 10240
NODES_PER_SC = N_PAD // N_SC
CHUNK_N = 128
SC_CHUNKS = NODES_PER_SC // CHUNK_N
GROUPS = CHUNK_N // LANES

MM_BLOCK = 1280


def _mm_body(x_ref, w_ref, bias_ref, a_ref, bp_ref):
    xb = x_ref[...]
    w1 = w_ref[:, :C]
    w2 = w_ref[:, C:]
    dn = (((1,), (1,)), ((), ()))
    a_ref[...] = (lax.dot_general(xb, w1 - w2, dn,
                                  preferred_element_type=jnp.float32)
                  + bias_ref[...]).astype(jnp.bfloat16)
    btf = lax.dot_general(w2, xb, dn, preferred_element_type=jnp.float32)
    bb = btf.astype(jnp.bfloat16)
    lo = lax.bitcast_convert_type(bb[:C // 2, :], jnp.uint16).astype(jnp.uint32)
    hi = lax.bitcast_convert_type(bb[C // 2:, :], jnp.uint16).astype(jnp.uint32)
    bp_ref[...] = lax.bitcast_convert_type(lo | (hi << 16), jnp.int32)


@jax.jit
def _mm_call(x, W, bias):
    grid = (N_PAD // MM_BLOCK,)
    return pl.pallas_call(
        _mm_body,
        grid=grid,
        in_specs=[
            pl.BlockSpec((MM_BLOCK, C), lambda i: (i, 0)),
            pl.BlockSpec((C, 2 * C), lambda i: (0, 0)),
            pl.BlockSpec((1, C), lambda i: (0, 0)),
        ],
        out_specs=[
            pl.BlockSpec((MM_BLOCK, C), lambda i: (i, 0)),
            pl.BlockSpec((C // 2, MM_BLOCK), lambda i: (0, i)),
        ],
        out_shape=[
            jax.ShapeDtypeStruct((N_PAD, C), jnp.bfloat16),
            jax.ShapeDtypeStruct((C // 2, N_PAD), jnp.int32),
        ],
    )(x, W, bias)


def _elu_body(a_ref, mt_ref, o_ref):
    w = lax.bitcast_convert_type(mt_ref[...], jnp.uint32)
    lo = lax.bitcast_convert_type(
        (w & 0xFFFF).astype(jnp.uint16), jnp.bfloat16).astype(jnp.float32)
    hi = lax.bitcast_convert_type(
        (w >> 16).astype(jnp.uint16), jnp.bfloat16).astype(jnp.float32)
    m = jnp.concatenate([lo, hi], axis=0)
    z = a_ref[...].astype(jnp.float32) + m.T
    o_ref[...] = jnp.where(z > 0, z, jnp.exp(z) - 1.0)


@jax.jit
def _elu_call(A, Mt):
    grid = (N_PAD // MM_BLOCK,)
    return pl.pallas_call(
        _elu_body,
        grid=grid,
        in_specs=[
            pl.BlockSpec((MM_BLOCK, C), lambda i: (i, 0)),
            pl.BlockSpec((C // 2, MM_BLOCK), lambda i: (0, i)),
        ],
        out_specs=pl.BlockSpec((MM_BLOCK, C), lambda i: (i, 0)),
        out_shape=jax.ShapeDtypeStruct((N_NODES, C), jnp.float32),
    )(A, Mt)


def _sc_body(bt_hbm, idx_hbm, mt_hbm, b_v,
             idx0, idx1, o0, o1, isem0, isem1, osem0, osem1):
    sc = lax.axis_index("c")
    tile = lax.axis_index("s")
    nbase = sc * NODES_PER_SC
    pltpu.sync_copy(bt_hbm.at[pl.ds(tile * (FPT // 2), FPT // 2)], b_v)

    idx_bufs = (idx0, idx1)
    idx_sems = (isem0, isem1)
    o_bufs = (o0, o1)
    o_sems = (osem0, osem1)

    def start_idx(ch, buf, sm):
        pltpu.make_async_copy(
            idx_hbm.at[:, pl.ds(nbase + ch * CHUNK_N, CHUNK_N)],
            buf, sm).start()

    def wait_idx(buf, sm):
        pltpu.make_async_copy(
            idx_hbm.at[:, pl.ds(0, CHUNK_N)], buf, sm).wait()

    def wait_store(buf, sm):
        pltpu.make_async_copy(
            buf, mt_hbm.at[pl.ds(0, FPT // 2), pl.ds(0, CHUNK_N)], sm).wait()

    iota = lax.iota(jnp.int32, LANES)

    def compute_chunk(idx_v, o_v):
        def group_body(g, carry):
            accs = [None] * (FPT // 2)
            for k in range(K):
                nid = idx_v[k, pl.ds(g * LANES, LANES)]
                for fp in range(FPT // 2):
                    w = plsc.load_gather(
                        b_v, [jnp.full((LANES,), fp, jnp.int32), nid])
                    v = plsc.bitcast(w, jnp.bfloat16)
                    accs[fp] = v if k == 0 else jnp.maximum(accs[fp], v)
            for fp in range(FPT // 2):
                o_v[fp, pl.ds(g * LANES, LANES)] = plsc.bitcast(
                    accs[fp], jnp.int32)
            return carry
        lax.fori_loop(0, GROUPS, group_body, 0)

    start_idx(0, idx_bufs[0], idx_sems[0])

    def pair_body(i, carry):
        base = i * 2
        for par in range(2):
            ch = base + par
            @pl.when(ch + 1 < SC_CHUNKS)
            def _():
                start_idx(ch + 1, idx_bufs[1 - par], idx_sems[1 - par])
            wait_idx(idx_bufs[par], idx_sems[par])
            @pl.when(ch >= 2)
            def _():
                wait_store(o_bufs[par], o_sems[par])
            compute_chunk(idx_bufs[par], o_bufs[par])
            pltpu.make_async_copy(
                o_bufs[par],
                mt_hbm.at[pl.ds(tile * (FPT // 2), FPT // 2),
                          pl.ds(nbase + ch * CHUNK_N, CHUNK_N)],
                o_sems[par]).start()
        return carry

    lax.fori_loop(0, SC_CHUNKS // 2, pair_body, 0)
    wait_store(o_bufs[0], o_sems[0])
    wait_store(o_bufs[1], o_sems[1])


@jax.jit
def _sc_call(Bt, idx):
    mesh = plsc.VectorSubcoreMesh(core_axis_name="c", subcore_axis_name="s")
    f = functools.partial(
        pl.kernel,
        out_type=jax.ShapeDtypeStruct((C // 2, N_PAD), jnp.int32),
        mesh=mesh,
        compiler_params=pltpu.CompilerParams(needs_layout_passes=False),
        scratch_types=(
            [pltpu.VMEM((FPT // 2, N_PAD), jnp.int32)]
            + [pltpu.VMEM((K, CHUNK_N), jnp.int32)] * 2
            + [pltpu.VMEM((FPT // 2, CHUNK_N), jnp.int32)] * 2
            + [pltpu.SemaphoreType.DMA] * 4
        ),
    )(_sc_body)
    return f(Bt, idx)


def kernel(x, edge_index, W, b):
    x = x.astype(jnp.float32)
    ei = edge_index.astype(jnp.int32)
    A, Bp = _mm_call(x, W, b.reshape(1, C))
    ei_pad = jnp.concatenate(
        [ei, jnp.zeros((N_PAD - N_NODES, K), jnp.int32)], axis=0)
    Mt = _sc_call(Bp, ei_pad.T)
    return _elu_call(A, Mt)
